# baseline (speedup 1.0000x reference)
import functools

import jax
import jax.numpy as jnp
from jax import lax
from jax.experimental import pallas as pl
from jax.experimental.pallas import tpu as pltpu
from jax.experimental.pallas import tpu_sc as plsc

N = 10000
D = 128
EMB = 128
OUT = 64
E = 320000

NPAD = 10240
NC = 2
NS = 16
NW = NC * NS
EPT = E // NW
CHUNK = 128
NCHUNK = 80
EPT_PAD = NCHUNK * CHUNK
ROWS_PER_TILE = NPAD // NS


def _encoder_body(x_ref, wd_ref, bd_ref, wg_ref, al_ref, ar_ref,
                  feat_ref, elr_ref):
    xb = x_ref[...]
    h = jnp.maximum(xb @ wd_ref[...] + bd_ref[...][None, :], 0.0)
    feat = h @ wg_ref[...]
    feat_ref[...] = feat
    el = jnp.sum(feat * al_ref[...][None, :], axis=1)
    er = jnp.sum(feat * ar_ref[...][None, :], axis=1)
    elr_ref[...] = jnp.stack([el, er])


def _encoder(x_pad, Wd, bd, Wg, attn_l, attn_r):
    blk = 512
    grid = (NPAD // blk,)
    return pl.pallas_call(
        _encoder_body,
        grid=grid,
        in_specs=[
            pl.BlockSpec((blk, D), lambda i: (i, 0)),
            pl.BlockSpec((D, EMB), lambda i: (0, 0)),
            pl.BlockSpec((EMB,), lambda i: (0,)),
            pl.BlockSpec((EMB, OUT), lambda i: (0, 0)),
            pl.BlockSpec((OUT,), lambda i: (0,)),
            pl.BlockSpec((OUT,), lambda i: (0,)),
        ],
        out_specs=[
            pl.BlockSpec((blk, OUT), lambda i: (i, 0)),
            pl.BlockSpec((2, blk), lambda i: (0, i)),
        ],
        out_shape=[
            jax.ShapeDtypeStruct((NPAD, OUT), jnp.float32),
            jax.ShapeDtypeStruct((2, NPAD), jnp.float32),
        ],
    )(x_pad, Wd, bd, Wg, attn_l, attn_r)


def _edge_body(feat_hbm, elr_hbm, src_hbm, dst_hbm, zn_hbm, zd_hbm,
               out_num, out_den,
               el_v, er_v, src_v, dst_v, a_v, wi0, wi1, wo0, wo1,
               num_sh, den_sh, g0, g1):
    cid = lax.axis_index("c")
    sid = lax.axis_index("s")
    wid = cid * NS + sid

    pltpu.sync_copy(elr_hbm.at[0], el_v)
    pltpu.sync_copy(elr_hbm.at[1], er_v)
    pltpu.sync_copy(src_hbm.at[wid], src_v)
    pltpu.sync_copy(dst_hbm.at[wid], dst_v)

    rbase = sid * ROWS_PER_TILE
    pltpu.sync_copy(zn_hbm.at[pl.ds(rbase, ROWS_PER_TILE)],
                    num_sh.at[pl.ds(rbase, ROWS_PER_TILE)])
    pltpu.sync_copy(zd_hbm.at[pl.ds(rbase, ROWS_PER_TILE)],
                    den_sh.at[pl.ds(rbase, ROWS_PER_TILE)])
    plsc.subcore_barrier()

    iota = lax.iota(jnp.int32, 16)

    pltpu.async_copy(feat_hbm.at[src_v.at[0]], wi0, g0)
    pltpu.async_copy(feat_hbm.at[src_v.at[1]], wi1, g1)

    def pair_body(i, _):
        for b, w_in, w_out, gs in ((0, wi0, wo0, g0), (1, wi1, wo1, g1)):
            c = 2 * i + b

            for v in range(CHUNK // 16):
                idx = src_v[c, pl.ds(v * 16, 16)]
                jdx = dst_v[c, pl.ds(v * 16, 16)]
                s = plsc.load_gather(el_v, [idx]) + plsc.load_gather(er_v, [jdx])
                s = jnp.where(s >= 0.0, s, 0.2 * s)
                av = jnp.exp(s)
                gidx = c * CHUNK + v * 16 + iota
                av = jnp.where(gidx < EPT, av, 0.0)
                a_v[b, pl.ds(v * 16, 16)] = av

            pltpu.make_async_copy(feat_hbm.at[src_v.at[c]], w_in, gs).wait()

            bful = jnp.full((16,), b, jnp.int32)
            for e in range(CHUNK):
                sc = plsc.load_gather(a_v, [bful, jnp.full((16,), e, jnp.int32)])
                for k in range(OUT // 16):
                    w_out[e, pl.ds(k * 16, 16)] = w_in[e, pl.ds(k * 16, 16)] * sc

            @pl.when(i < NCHUNK // 2 - 1)
            def _():
                pltpu.async_copy(feat_hbm.at[src_v.at[c + 2]], w_in, gs)

            pltpu.sync_copy(w_out, num_sh.at[dst_v.at[c]], add=True)
            pltpu.sync_copy(a_v.at[b], den_sh.at[dst_v.at[c]], add=True)
        return 0

    lax.fori_loop(0, NCHUNK // 2, pair_body, 0)
    plsc.subcore_barrier()

    pltpu.sync_copy(num_sh.at[pl.ds(rbase, ROWS_PER_TILE)],
                    out_num.at[cid, pl.ds(rbase, ROWS_PER_TILE)])
    pltpu.sync_copy(den_sh.at[pl.ds(rbase, ROWS_PER_TILE)],
                    out_den.at[cid, pl.ds(rbase, ROWS_PER_TILE)])


def _edge_aggregate(feat, elr, src_p, dst_p, zn, zd):
    mesh = plsc.VectorSubcoreMesh(core_axis_name="c", subcore_axis_name="s",
                                  num_cores=NC, num_subcores=NS)
    fn = pl.kernel(
        _edge_body,
        out_type=[
            jax.ShapeDtypeStruct((NC, NPAD, OUT), jnp.float32),
            jax.ShapeDtypeStruct((NC, NPAD), jnp.float32),
        ],
        mesh=mesh,
        scratch_types=[
            pltpu.VMEM((NPAD,), jnp.float32),
            pltpu.VMEM((NPAD,), jnp.float32),
            pltpu.VMEM((NCHUNK, CHUNK), jnp.int32),
            pltpu.VMEM((NCHUNK, CHUNK), jnp.int32),
            pltpu.VMEM((2, CHUNK), jnp.float32),
            pltpu.VMEM((CHUNK, OUT), jnp.float32),
            pltpu.VMEM((CHUNK, OUT), jnp.float32),
            pltpu.VMEM((CHUNK, OUT), jnp.float32),
            pltpu.VMEM((CHUNK, OUT), jnp.float32),
            pltpu.VMEM_SHARED((NPAD, OUT), jnp.float32),
            pltpu.VMEM_SHARED((NPAD,), jnp.float32),
        ] + [pltpu.SemaphoreType.DMA] * 2,
        compiler_params=pltpu.CompilerParams(needs_layout_passes=False, use_tc_tiling_on_sc=False),
    )
    return fn(feat, elr, src_p, dst_p, zn, zd)


def _combine_body(num_ref, den_ref, bg_ref, emb_ref):
    num = num_ref[0] + num_ref[1]
    den = den_ref[0] + den_ref[1]
    emb_ref[...] = num / (den[:, None] + 1e-9) + bg_ref[...][None, :]


def _combine(num, den, bg):
    return pl.pallas_call(
        _combine_body,
        out_shape=jax.ShapeDtypeStruct((NPAD, OUT), jnp.float32),
    )(num, den, bg)


def _attr_body(x_ref, w1_ref, b1_ref, w2_ref, b2_ref, attr_ref):
    t = lax.dot_general(x_ref[...], w1_ref[...], (((0,), (0,)), ((), ())))
    t = jnp.maximum(t + b1_ref[...][None, :], 0.0)
    attr_ref[...] = t @ w2_ref[...] + b2_ref[...][None, :]


def _attr(x, W1, b1, W2, b2):
    return pl.pallas_call(
        _attr_body,
        out_shape=jax.ShapeDtypeStruct((D, OUT), jnp.float32),
    )(x, W1, b1, W2, b2)


def _xhat_body(emb_ref, attr_ref, out_ref):
    e = emb_ref[pl.ds(0, N), :]
    out_ref[...] = lax.dot_general(e, attr_ref[...], (((1,), (1,)), ((), ())))


def _xhat(embed, attr):
    return pl.pallas_call(
        _xhat_body,
        out_shape=jax.ShapeDtypeStruct((N, D), jnp.float32),
    )(embed, attr)


def _ahat_body(ei_ref, ej_ref, out_ref):
    s = lax.dot_general(ei_ref[...], ej_ref[...], (((1,), (1,)), ((), ())))
    out_ref[...] = 1.0 / (1.0 + jnp.exp(-s))


def _ahat(embed):
    bi, bj = 256, 2048
    return pl.pallas_call(
        _ahat_body,
        grid=(NPAD // bi, NPAD // bj),
        in_specs=[
            pl.BlockSpec((bi, OUT), lambda i, j: (i, 0)),
            pl.BlockSpec((bj, OUT), lambda i, j: (j, 0)),
        ],
        out_specs=pl.BlockSpec((bi, bj), lambda i, j: (i, j)),
        out_shape=jax.ShapeDtypeStruct((N, N), jnp.float32),
    )(embed, embed)


def kernel(x, edge_index, Wd, bd, Wg, attn_l, attn_r, bg, W1, b1, W2, b2):
    x_pad = jnp.concatenate(
        [x, jnp.zeros((NPAD - N, D), jnp.float32)], axis=0)

    src = edge_index[0].reshape(NW, EPT)
    dst = edge_index[1].reshape(NW, EPT)
    pad = jnp.full((NW, EPT_PAD - EPT), N, jnp.int32)
    src_p = jnp.concatenate([src, pad], axis=1).reshape(NW, NCHUNK, CHUNK)
    dst_p = jnp.concatenate([dst, pad], axis=1).reshape(NW, NCHUNK, CHUNK)

    zn = jnp.zeros((NPAD, OUT), jnp.float32)
    zd = jnp.zeros((NPAD,), jnp.float32)

    feat, elr = _encoder(x_pad, Wd, bd, Wg, attn_l, attn_r)
    num, den = _edge_aggregate(feat, elr, src_p, dst_p, zn, zd)
    embed = _combine(num, den, bg)
    attr = _attr(x, W1, b1, W2, b2)
    X_hat = _xhat(embed, attr)
    A_hat = _ahat(embed)
    return (A_hat, X_hat)


# speedup vs baseline: 1.1345x; 1.1345x over previous
import functools

import jax
import jax.numpy as jnp
from jax import lax
from jax.experimental import pallas as pl
from jax.experimental.pallas import tpu as pltpu
from jax.experimental.pallas import tpu_sc as plsc

N = 10000
D = 128
EMB = 128
OUT = 64
E = 320000

NPAD = 10240
NC = 2
NS = 16
NW = NC * NS
EPT = E // NW
CHUNK = 128
NCHUNK = 80
EPT_PAD = NCHUNK * CHUNK
ROWS_PER_TILE = NPAD // NS


def _encoder_body(x_ref, wd_ref, bd_ref, wg_ref, al_ref, ar_ref,
                  feat_ref, elr_ref):
    xb = x_ref[...]
    h = jnp.maximum(xb @ wd_ref[...] + bd_ref[...][None, :], 0.0)
    feat = h @ wg_ref[...]
    feat_ref[...] = feat
    el = jnp.sum(feat * al_ref[...][None, :], axis=1)
    er = jnp.sum(feat * ar_ref[...][None, :], axis=1)
    elr_ref[...] = jnp.stack([el, er])


def _encoder(x_pad, Wd, bd, Wg, attn_l, attn_r):
    blk = 512
    grid = (NPAD // blk,)
    return pl.pallas_call(
        _encoder_body,
        grid=grid,
        in_specs=[
            pl.BlockSpec((blk, D), lambda i: (i, 0)),
            pl.BlockSpec((D, EMB), lambda i: (0, 0)),
            pl.BlockSpec((EMB,), lambda i: (0,)),
            pl.BlockSpec((EMB, OUT), lambda i: (0, 0)),
            pl.BlockSpec((OUT,), lambda i: (0,)),
            pl.BlockSpec((OUT,), lambda i: (0,)),
        ],
        out_specs=[
            pl.BlockSpec((blk, OUT), lambda i: (i, 0)),
            pl.BlockSpec((2, blk), lambda i: (0, i)),
        ],
        out_shape=[
            jax.ShapeDtypeStruct((NPAD, OUT), jnp.float32),
            jax.ShapeDtypeStruct((2, NPAD), jnp.float32),
        ],
    )(x_pad, Wd, bd, Wg, attn_l, attn_r)


def _edge_body(feat_hbm, elr_hbm, src_hbm, dst_hbm, zn_hbm, zd_hbm,
               out_num, out_den,
               el_v, er_v, src_v, dst_v, a_v, wi0, wi1, wo0, wo1,
               num_sh, den_sh, g0, g1):
    cid = lax.axis_index("c")
    sid = lax.axis_index("s")
    wid = cid * NS + sid

    pltpu.sync_copy(elr_hbm.at[0], el_v)
    pltpu.sync_copy(elr_hbm.at[1], er_v)
    pltpu.sync_copy(src_hbm.at[wid], src_v)
    pltpu.sync_copy(dst_hbm.at[wid], dst_v)

    rbase = sid * ROWS_PER_TILE
    pltpu.sync_copy(zn_hbm.at[pl.ds(rbase, ROWS_PER_TILE)],
                    num_sh.at[pl.ds(rbase, ROWS_PER_TILE)])
    pltpu.sync_copy(zd_hbm.at[pl.ds(rbase, ROWS_PER_TILE)],
                    den_sh.at[pl.ds(rbase, ROWS_PER_TILE)])
    plsc.subcore_barrier()

    iota = lax.iota(jnp.int32, 16)

    pltpu.async_copy(feat_hbm.at[src_v.at[0]], wi0, g0)
    pltpu.async_copy(feat_hbm.at[src_v.at[1]], wi1, g1)

    def pair_body(i, _):
        for b, w_in, w_out, gs in ((0, wi0, wo0, g0), (1, wi1, wo1, g1)):
            c = 2 * i + b

            avs = []
            for v in range(CHUNK // 16):
                idx = src_v[c, pl.ds(v * 16, 16)]
                jdx = dst_v[c, pl.ds(v * 16, 16)]
                s = plsc.load_gather(el_v, [idx]) + plsc.load_gather(er_v, [jdx])
                s = jnp.where(s >= 0.0, s, 0.2 * s)
                av = jnp.exp(s)
                gidx = c * CHUNK + v * 16 + iota
                av = jnp.where(gidx < EPT, av, 0.0)
                a_v[b, pl.ds(v * 16, 16)] = av
                avs.append(av)

            pltpu.make_async_copy(feat_hbm.at[src_v.at[c]], w_in, gs).wait()

            for v in range(CHUNK // 16):
                for j in range(16):
                    e = v * 16 + j
                    sc = jnp.take_along_axis(
                        avs[v], jnp.full((16,), j, jnp.int32), axis=0)
                    for k in range(OUT // 16):
                        w_out[e, pl.ds(k * 16, 16)] = w_in[e, pl.ds(k * 16, 16)] * sc

            @pl.when(i < NCHUNK // 2 - 1)
            def _():
                pltpu.async_copy(feat_hbm.at[src_v.at[c + 2]], w_in, gs)

            pltpu.sync_copy(w_out, num_sh.at[dst_v.at[c]], add=True)
            pltpu.sync_copy(a_v.at[b], den_sh.at[dst_v.at[c]], add=True)
        return 0

    lax.fori_loop(0, NCHUNK // 2, pair_body, 0)
    plsc.subcore_barrier()

    pltpu.sync_copy(num_sh.at[pl.ds(rbase, ROWS_PER_TILE)],
                    out_num.at[cid, pl.ds(rbase, ROWS_PER_TILE)])
    pltpu.sync_copy(den_sh.at[pl.ds(rbase, ROWS_PER_TILE)],
                    out_den.at[cid, pl.ds(rbase, ROWS_PER_TILE)])


def _edge_aggregate(feat, elr, src_p, dst_p, zn, zd):
    mesh = plsc.VectorSubcoreMesh(core_axis_name="c", subcore_axis_name="s",
                                  num_cores=NC, num_subcores=NS)
    fn = pl.kernel(
        _edge_body,
        out_type=[
            jax.ShapeDtypeStruct((NC, NPAD, OUT), jnp.float32),
            jax.ShapeDtypeStruct((NC, NPAD), jnp.float32),
        ],
        mesh=mesh,
        scratch_types=[
            pltpu.VMEM((NPAD,), jnp.float32),
            pltpu.VMEM((NPAD,), jnp.float32),
            pltpu.VMEM((NCHUNK, CHUNK), jnp.int32),
            pltpu.VMEM((NCHUNK, CHUNK), jnp.int32),
            pltpu.VMEM((2, CHUNK), jnp.float32),
            pltpu.VMEM((CHUNK, OUT), jnp.float32),
            pltpu.VMEM((CHUNK, OUT), jnp.float32),
            pltpu.VMEM((CHUNK, OUT), jnp.float32),
            pltpu.VMEM((CHUNK, OUT), jnp.float32),
            pltpu.VMEM_SHARED((NPAD, OUT), jnp.float32),
            pltpu.VMEM_SHARED((NPAD,), jnp.float32),
        ] + [pltpu.SemaphoreType.DMA] * 2,
        compiler_params=pltpu.CompilerParams(needs_layout_passes=False, use_tc_tiling_on_sc=False),
    )
    return fn(feat, elr, src_p, dst_p, zn, zd)


def _combine_body(num_ref, den_ref, bg_ref, emb_ref):
    num = num_ref[0] + num_ref[1]
    den = den_ref[0] + den_ref[1]
    emb_ref[...] = num / (den[:, None] + 1e-9) + bg_ref[...][None, :]


def _combine(num, den, bg):
    return pl.pallas_call(
        _combine_body,
        out_shape=jax.ShapeDtypeStruct((NPAD, OUT), jnp.float32),
    )(num, den, bg)


def _attr_body(x_ref, w1_ref, b1_ref, w2_ref, b2_ref, attr_ref):
    t = lax.dot_general(x_ref[...], w1_ref[...], (((0,), (0,)), ((), ())))
    t = jnp.maximum(t + b1_ref[...][None, :], 0.0)
    attr_ref[...] = t @ w2_ref[...] + b2_ref[...][None, :]


def _attr(x, W1, b1, W2, b2):
    return pl.pallas_call(
        _attr_body,
        out_shape=jax.ShapeDtypeStruct((D, OUT), jnp.float32),
    )(x, W1, b1, W2, b2)


def _xhat_body(emb_ref, attr_ref, out_ref):
    e = emb_ref[pl.ds(0, N), :]
    out_ref[...] = lax.dot_general(e, attr_ref[...], (((1,), (1,)), ((), ())))


def _xhat(embed, attr):
    return pl.pallas_call(
        _xhat_body,
        out_shape=jax.ShapeDtypeStruct((N, D), jnp.float32),
    )(embed, attr)


def _ahat_body(ei_ref, ej_ref, out_ref):
    s = lax.dot_general(ei_ref[...], ej_ref[...], (((1,), (1,)), ((), ())))
    out_ref[...] = 1.0 / (1.0 + jnp.exp(-s))


def _ahat(embed):
    bi, bj = 256, 2048
    return pl.pallas_call(
        _ahat_body,
        grid=(NPAD // bi, NPAD // bj),
        in_specs=[
            pl.BlockSpec((bi, OUT), lambda i, j: (i, 0)),
            pl.BlockSpec((bj, OUT), lambda i, j: (j, 0)),
        ],
        out_specs=pl.BlockSpec((bi, bj), lambda i, j: (i, j)),
        out_shape=jax.ShapeDtypeStruct((N, N), jnp.float32),
    )(embed, embed)


def kernel(x, edge_index, Wd, bd, Wg, attn_l, attn_r, bg, W1, b1, W2, b2):
    x_pad = jnp.concatenate(
        [x, jnp.zeros((NPAD - N, D), jnp.float32)], axis=0)

    src = edge_index[0].reshape(NW, EPT)
    dst = edge_index[1].reshape(NW, EPT)
    pad = jnp.full((NW, EPT_PAD - EPT), N, jnp.int32)
    src_p = jnp.concatenate([src, pad], axis=1).reshape(NW, NCHUNK, CHUNK)
    dst_p = jnp.concatenate([dst, pad], axis=1).reshape(NW, NCHUNK, CHUNK)

    zn = jnp.zeros((NPAD, OUT), jnp.float32)
    zd = jnp.zeros((NPAD,), jnp.float32)

    feat, elr = _encoder(x_pad, Wd, bd, Wg, attn_l, attn_r)
    num, den = _edge_aggregate(feat, elr, src_p, dst_p, zn, zd)
    embed = _combine(num, den, bg)
    attr = _attr(x, W1, b1, W2, b2)
    X_hat = _xhat(embed, attr)
    A_hat = _ahat(embed)
    return (A_hat, X_hat)


# speedup vs baseline: 1.3183x; 1.1620x over previous
import functools

import jax
import jax.numpy as jnp
from jax import lax
from jax.experimental import pallas as pl
from jax.experimental.pallas import tpu as pltpu
from jax.experimental.pallas import tpu_sc as plsc

N = 10000
D = 128
EMB = 128
OUT = 64
E = 320000

NPAD = 10240
NC = 2
NS = 16
NW = NC * NS
EPT = E // NW
CHUNK = 128
NCHUNK = 80
EPT_PAD = NCHUNK * CHUNK
ROWS_PER_TILE = NPAD // NS


def _encoder_body(x_ref, wd_ref, bd_ref, wg_ref, al_ref, ar_ref,
                  feat_ref, elr_ref):
    xb = x_ref[...]
    h = jnp.maximum(xb @ wd_ref[...] + bd_ref[...][None, :], 0.0)
    feat = h @ wg_ref[...]
    feat_ref[...] = feat
    el = jnp.sum(feat * al_ref[...][None, :], axis=1)
    er = jnp.sum(feat * ar_ref[...][None, :], axis=1)
    elr_ref[...] = jnp.stack([el, er])


def _encoder(x_pad, Wd, bd, Wg, attn_l, attn_r):
    blk = 512
    grid = (NPAD // blk,)
    return pl.pallas_call(
        _encoder_body,
        grid=grid,
        in_specs=[
            pl.BlockSpec((blk, D), lambda i: (i, 0)),
            pl.BlockSpec((D, EMB), lambda i: (0, 0)),
            pl.BlockSpec((EMB,), lambda i: (0,)),
            pl.BlockSpec((EMB, OUT), lambda i: (0, 0)),
            pl.BlockSpec((OUT,), lambda i: (0,)),
            pl.BlockSpec((OUT,), lambda i: (0,)),
        ],
        out_specs=[
            pl.BlockSpec((blk, OUT), lambda i: (i, 0)),
            pl.BlockSpec((2, blk), lambda i: (0, i)),
        ],
        out_shape=[
            jax.ShapeDtypeStruct((NPAD, OUT), jnp.float32),
            jax.ShapeDtypeStruct((2, NPAD), jnp.float32),
        ],
    )(x_pad, Wd, bd, Wg, attn_l, attn_r)


def _edge_body(feat_hbm, elr_hbm, src_hbm, dst_hbm, zn_hbm, zd_hbm,
               out_num, out_den,
               el_v, er_v, src_v, dst_v, a_v, wi0, wi1, wo0, wo1,
               num_sh, den_sh, g0, g1):
    cid = lax.axis_index("c")
    sid = lax.axis_index("s")
    wid = cid * NS + sid

    pltpu.sync_copy(elr_hbm.at[0], el_v)
    pltpu.sync_copy(elr_hbm.at[1], er_v)
    pltpu.sync_copy(src_hbm.at[wid], src_v)
    pltpu.sync_copy(dst_hbm.at[wid], dst_v)

    rbase = sid * ROWS_PER_TILE
    pltpu.sync_copy(zn_hbm.at[pl.ds(rbase, ROWS_PER_TILE)],
                    num_sh.at[pl.ds(rbase, ROWS_PER_TILE)])
    pltpu.sync_copy(zd_hbm.at[pl.ds(rbase, ROWS_PER_TILE)],
                    den_sh.at[pl.ds(rbase, ROWS_PER_TILE)])
    plsc.subcore_barrier()

    iota = lax.iota(jnp.int32, 16)

    pltpu.async_copy(feat_hbm.at[src_v.at[0]], wi0, g0)
    pltpu.async_copy(feat_hbm.at[src_v.at[1]], wi1, g1)

    def pair_body(i, _):
        for b, w_in, w_out, gs in ((0, wi0, wo0, g0), (1, wi1, wo1, g1)):
            c = 2 * i + b

            avs = []
            for v in range(CHUNK // 16):
                idx = src_v[c, pl.ds(v * 16, 16)]
                jdx = dst_v[c, pl.ds(v * 16, 16)]
                s = plsc.load_gather(el_v, [idx]) + plsc.load_gather(er_v, [jdx])
                s = jnp.where(s >= 0.0, s, 0.2 * s)
                av = jnp.exp(s)
                gidx = c * CHUNK + v * 16 + iota
                av = jnp.where(gidx < EPT, av, 0.0)
                a_v[b, pl.ds(v * 16, 16)] = av
                avs.append(av)

            pltpu.make_async_copy(feat_hbm.at[src_v.at[c]], w_in, gs).wait()

            for v in range(CHUNK // 16):
                for j in range(16):
                    e = v * 16 + j
                    sc = jnp.take_along_axis(
                        avs[v], jnp.full((16,), j, jnp.int32), axis=0)
                    for k in range(OUT // 16):
                        w_out[e, pl.ds(k * 16, 16)] = w_in[e, pl.ds(k * 16, 16)] * sc

            @pl.when(i < NCHUNK // 2 - 1)
            def _():
                pltpu.async_copy(feat_hbm.at[src_v.at[c + 2]], w_in, gs)

            pltpu.sync_copy(w_out, num_sh.at[dst_v.at[c]], add=True)
            pltpu.sync_copy(a_v.at[b], den_sh.at[dst_v.at[c]], add=True)
        return 0

    lax.fori_loop(0, NCHUNK // 2, pair_body, 0)
    plsc.subcore_barrier()

    pltpu.sync_copy(num_sh.at[pl.ds(rbase, ROWS_PER_TILE)],
                    out_num.at[cid, pl.ds(rbase, ROWS_PER_TILE)])
    pltpu.sync_copy(den_sh.at[pl.ds(rbase, ROWS_PER_TILE)],
                    out_den.at[cid, pl.ds(rbase, ROWS_PER_TILE)])


def _edge_aggregate(feat, elr, src_p, dst_p, zn, zd):
    mesh = plsc.VectorSubcoreMesh(core_axis_name="c", subcore_axis_name="s",
                                  num_cores=NC, num_subcores=NS)
    fn = pl.kernel(
        _edge_body,
        out_type=[
            jax.ShapeDtypeStruct((NC, NPAD, OUT), jnp.float32),
            jax.ShapeDtypeStruct((NC, NPAD), jnp.float32),
        ],
        mesh=mesh,
        scratch_types=[
            pltpu.VMEM((NPAD,), jnp.float32),
            pltpu.VMEM((NPAD,), jnp.float32),
            pltpu.VMEM((NCHUNK, CHUNK), jnp.int32),
            pltpu.VMEM((NCHUNK, CHUNK), jnp.int32),
            pltpu.VMEM((2, CHUNK), jnp.float32),
            pltpu.VMEM((CHUNK, OUT), jnp.float32),
            pltpu.VMEM((CHUNK, OUT), jnp.float32),
            pltpu.VMEM((CHUNK, OUT), jnp.float32),
            pltpu.VMEM((CHUNK, OUT), jnp.float32),
            pltpu.VMEM_SHARED((NPAD, OUT), jnp.float32),
            pltpu.VMEM_SHARED((NPAD,), jnp.float32),
        ] + [pltpu.SemaphoreType.DMA] * 2,
        compiler_params=pltpu.CompilerParams(needs_layout_passes=False, use_tc_tiling_on_sc=False),
    )
    return fn(feat, elr, src_p, dst_p, zn, zd)


def _post_body(num_ref, den_ref, bg_ref, x_ref, w1_ref, b1_ref, w2_ref,
               b2_ref, emb_ref, xhat_ref):
    num = num_ref[0] + num_ref[1]
    den = den_ref[0] + den_ref[1]
    embed = num / (den[:, None] + 1e-9) + bg_ref[...][None, :]
    emb_ref[...] = embed
    t = lax.dot_general(x_ref[...], w1_ref[...], (((0,), (0,)), ((), ())))
    t = jnp.maximum(t + b1_ref[...][None, :], 0.0)
    attr = t @ w2_ref[...] + b2_ref[...][None, :]
    xhat_ref[...] = lax.dot_general(
        embed[:N], attr, (((1,), (1,)), ((), ())))


def _post(num, den, bg, x, W1, b1, W2, b2):
    return pl.pallas_call(
        _post_body,
        out_shape=[
            jax.ShapeDtypeStruct((NPAD, OUT), jnp.float32),
            jax.ShapeDtypeStruct((N, D), jnp.float32),
        ],
    )(num, den, bg, x, W1, b1, W2, b2)


def _ahat_body(ei_ref, ej_ref, out_ref):
    s = lax.dot_general(ei_ref[...], ej_ref[...], (((1,), (1,)), ((), ())))
    out_ref[...] = 1.0 / (1.0 + jnp.exp(-s))


def _ahat(embed):
    bi, bj = 512, 2048
    return pl.pallas_call(
        _ahat_body,
        grid=(NPAD // bi, NPAD // bj),
        in_specs=[
            pl.BlockSpec((bi, OUT), lambda i, j: (i, 0)),
            pl.BlockSpec((bj, OUT), lambda i, j: (j, 0)),
        ],
        out_specs=pl.BlockSpec((bi, bj), lambda i, j: (i, j)),
        out_shape=jax.ShapeDtypeStruct((N, N), jnp.float32),
    )(embed, embed)


def kernel(x, edge_index, Wd, bd, Wg, attn_l, attn_r, bg, W1, b1, W2, b2):
    x_pad = jnp.concatenate(
        [x, jnp.zeros((NPAD - N, D), jnp.float32)], axis=0)

    src = edge_index[0].reshape(NW, EPT)
    dst = edge_index[1].reshape(NW, EPT)
    pad = jnp.full((NW, EPT_PAD - EPT), N, jnp.int32)
    src_p = jnp.concatenate([src, pad], axis=1).reshape(NW, NCHUNK, CHUNK)
    dst_p = jnp.concatenate([dst, pad], axis=1).reshape(NW, NCHUNK, CHUNK)

    zn = jnp.zeros((NPAD, OUT), jnp.float32)
    zd = jnp.zeros((NPAD,), jnp.float32)

    feat, elr = _encoder(x_pad, Wd, bd, Wg, attn_l, attn_r)
    num, den = _edge_aggregate(feat, elr, src_p, dst_p, zn, zd)
    embed, X_hat = _post(num, den, bg, x, W1, b1, W2, b2)
    A_hat = _ahat(embed)
    return (A_hat, X_hat)
